# baseline (speedup 1.0000x reference)
import functools

import jax
import jax.numpy as jnp
from jax import lax
from jax.experimental import pallas as pl
from jax.experimental.pallas import tpu as pltpu
from jax.experimental.pallas import tpu_sc as plsc

NUM_ROBOT = 2048
FEAT_DIM = 32
K_NBR = 64


def _embed(feat, emb, pid_full):
    B, N = feat.shape
    F = emb.shape[1]

    def body(f_ref, emb_ref, pid_ref, o_ref):
        f = f_ref[0]
        e = emb_ref[...]
        x = pid_ref[...]
        for c in range(3):
            x = x + jnp.where(f == c, 1.0, 0.0) * e[c:c + 1, :]
        o_ref[0] = x

    return pl.pallas_call(
        body,
        grid=(B,),
        in_specs=[
            pl.BlockSpec((1, N, 1), lambda b: (b, 0, 0)),
            pl.BlockSpec((3, F), lambda b: (0, 0)),
            pl.BlockSpec((N, F), lambda b: (0, 0)),
        ],
        out_specs=pl.BlockSpec((1, N, F), lambda b: (b, 0, 0)),
        out_shape=jax.ShapeDtypeStruct((B, N, F), jnp.float32),
    )(feat[..., None], emb, pid_full)


def _fps(px, py, pz, n_s):
    B, R, C = px.shape
    N = R * C
    CS = min(n_s, 128)
    RS = n_s // CS

    def body(pall_ref, qx_ref, qy_ref, qz_ref):
        pall = pall_ref[...]
        pxv = pall[:B]
        pyv = pall[B:2 * B]
        pzv = pall[2 * B:]
        flat = (lax.broadcasted_iota(jnp.int32, (3 * B, R, C), 1) * C
                + lax.broadcasted_iota(jnp.int32, (3 * B, R, C), 2))
        flat_s = (lax.broadcasted_iota(jnp.int32, (B, RS, CS), 1) * CS
                  + lax.broadcasted_iota(jnp.int32, (B, RS, CS), 2))

        def coords(last):
            m = flat == jnp.concatenate([last, last, last], axis=0)
            qall = jnp.sum(jnp.where(m, pall, 0.0), axis=(1, 2),
                           keepdims=True)
            return qall[:B], qall[B:2 * B], qall[2 * B:]

        def write_q(slot, qxs, qys, qzs):
            sm = flat_s == slot
            qx_ref[...] = jnp.where(sm, qxs, qx_ref[...])
            qy_ref[...] = jnp.where(sm, qys, qy_ref[...])
            qz_ref[...] = jnp.where(sm, qzs, qz_ref[...])

        def it(i, carry):
            dists, last = carry
            qxs, qys, qzs = coords(last)
            write_q(i - 1, qxs, qys, qzs)
            d = (pxv - qxs) ** 2 + (pyv - qys) ** 2 + (pzv - qzs) ** 2
            dists = jnp.minimum(dists, d)
            mx = jnp.max(dists, axis=(1, 2), keepdims=True)
            nxt = jnp.min(jnp.where(dists == mx, flat[:B], N),
                          axis=(1, 2), keepdims=True)
            return dists, nxt

        dists0 = jnp.full((B, R, C), jnp.inf, jnp.float32)
        last0 = jnp.zeros((B, 1, 1), jnp.int32)
        _, last = lax.fori_loop(1, n_s, it, (dists0, last0))
        qxs, qys, qzs = coords(last)
        write_q(n_s - 1, qxs, qys, qzs)

    out = jax.ShapeDtypeStruct((B, RS, CS), jnp.float32)
    return pl.pallas_call(
        body,
        out_shape=[out, out, out],
    )(jnp.concatenate([px, py, pz], axis=0))


def _matbias(a, w, b):
    M, Kd = a.shape
    H = w.shape[1]
    RB = min(M, 2048)

    def body(a_ref, w_ref, b_ref, o_ref):
        o_ref[...] = (jnp.dot(a_ref[...], w_ref[...],
                              preferred_element_type=jnp.float32)
                      + b_ref[...])

    return pl.pallas_call(
        body,
        grid=(M // RB,),
        in_specs=[
            pl.BlockSpec((RB, Kd), lambda i: (i, 0)),
            pl.BlockSpec((Kd, H), lambda i: (0, 0)),
            pl.BlockSpec((1, H), lambda i: (0, 0)),
        ],
        out_specs=pl.BlockSpec((RB, H), lambda i: (i, 0)),
        out_shape=jax.ShapeDtypeStruct((M, H), jnp.float32),
    )(a, w, b[None, :])


def _select(px, py, pz, qx, qy, qz, r2, idx_mod, Q=64):
    B, N = px.shape
    n_s = qx.shape[1]
    Q = min(Q, n_s)
    nblk = n_s // Q
    K = K_NBR

    def body(px_ref, py_ref, pz_ref, qx_ref, qy_ref, qz_ref, nbr_ref, val_ref):
        b = pl.program_id(0)
        pxv = px_ref[0]
        pyv = py_ref[0]
        pzv = pz_ref[0]
        qxv = qx_ref[0]
        qyv = qy_ref[0]
        qzv = qz_ref[0]
        d2 = (qxv - pxv) ** 2 + (qyv - pyv) ** 2 + (qzv - pzv) ** 2
        d2 = jnp.where(d2 <= r2, d2, jnp.inf)
        lane = lax.broadcasted_iota(jnp.int32, (Q, N), 1)
        col = lax.broadcasted_iota(jnp.int32, (Q, K), 1)
        cnt = jnp.sum(jnp.where(d2 < jnp.inf, 1, 0), axis=1)
        kmax = jnp.minimum(jnp.max(cnt), K)

        def it(k, carry):
            d2c, nbr, val = carry
            m = jnp.min(d2c, axis=1, keepdims=True)
            idx = jnp.min(jnp.where(d2c == m, lane, N), axis=1, keepdims=True)
            ok = m < jnp.inf
            nbr = jnp.where(col == k, idx, nbr)
            val = jnp.where(col == k, jnp.where(ok, 1.0, 0.0), val)
            d2c = jnp.where(lane == idx, jnp.inf, d2c)
            return d2c, nbr, val

        nbr0 = jnp.zeros((Q, K), jnp.int32)
        val0 = jnp.zeros((Q, K), jnp.float32)
        _, nbr, val = lax.fori_loop(0, kmax, it, (d2, nbr0, val0))
        nbr_ref[0] = nbr + (b % idx_mod) * N
        val_ref[0] = val

    return pl.pallas_call(
        body,
        grid=(B, nblk),
        in_specs=[pl.BlockSpec((1, 1, N), lambda b, q: (b, 0, 0))] * 3
        + [pl.BlockSpec((1, Q, 1), lambda b, q: (b, q, 0))] * 3,
        out_specs=[
            pl.BlockSpec((1, Q, K), lambda b, q: (b, q, 0)),
            pl.BlockSpec((1, Q, K), lambda b, q: (b, q, 0)),
        ],
        out_shape=[
            jax.ShapeDtypeStruct((B, n_s, K), jnp.int32),
            jax.ShapeDtypeStruct((B, n_s, K), jnp.float32),
        ],
    )(px[:, None, :], py[:, None, :], pz[:, None, :],
      qx[..., None], qy[..., None], qz[..., None])


def _sc_gather(table, idx, split):
    V, D = table.shape
    E = idx.shape[0]
    NW = 32
    NB = 2
    epw = E // NW
    chunk = min(epw // NB, 128, max(8, (64 * 1024) // (D * 4)))
    nchunk = epw // chunk
    assert nchunk % NB == 0
    Vs = V // split
    stage = split == 2
    mesh = plsc.VectorSubcoreMesh(core_axis_name="c", subcore_axis_name="s")

    scratch = [
        pltpu.VMEM((epw,), jnp.int32),
        [pltpu.VMEM((chunk, D), jnp.float32) for _ in range(NB)],
        [pltpu.SemaphoreType.DMA for _ in range(NB)],
    ]
    if stage:
        scratch = [pltpu.VMEM_SHARED((Vs, D), jnp.float32)] + scratch

    @functools.partial(
        pl.kernel,
        mesh=mesh,
        out_type=jax.ShapeDtypeStruct((E, D), jnp.float32),
        scratch_types=scratch,
    )
    def k(table_hbm, idx_hbm, out_hbm, *refs):
        if stage:
            sp, idx_v, bufs, sems = refs
        else:
            idx_v, bufs, sems = refs
            sp = table_hbm
        ci = lax.axis_index("c")
        si = lax.axis_index("s")
        if split == 2:
            base = ci * (E // 2) + si * epw
        else:
            base = (si * 2 + ci) * epw

        if stage:
            @pl.when(si == 0)
            def _():
                pltpu.sync_copy(table_hbm.at[pl.ds(ci * Vs, Vs)], sp)

        pltpu.sync_copy(idx_hbm.at[pl.ds(base, epw)], idx_v)
        if stage:
            plsc.subcore_barrier()

        def start(c, b):
            off = pl.multiple_of(c * chunk, 8)
            pltpu.async_copy(sp.at[idx_v.at[pl.ds(off, chunk)]],
                             bufs[b], sems[b])

        def finish(c, b):
            off = pl.multiple_of(c * chunk, 8)
            pltpu.make_async_copy(sp.at[idx_v.at[pl.ds(off, chunk)]],
                                  bufs[b], sems[b]).wait()
            off = pl.multiple_of(base + c * chunk, 8)
            pltpu.sync_copy(bufs[b], out_hbm.at[pl.ds(off, chunk)])

        for b in range(NB):
            start(b, b)

        @pl.loop(NB, nchunk, step=NB)
        def _(c0):
            for b in range(NB):
                finish(c0 - NB + b, b)
                start(c0 + b, b)

        for b in range(NB):
            finish(nchunk - NB + b, b)

    return k(table, idx)


def _mlpmax(G, wq, val, w2, b2, w3, b3, Qc=16):
    B, n_s, H1 = wq.shape
    K = val.shape[2]
    H2 = w2.shape[1]
    Ho = w3.shape[1]
    Qc = min(Qc, n_s)
    nblk = n_s // Qc

    def body(g_ref, wq_ref, v_ref, w2_ref, b2_ref, w3_ref, b3_ref, o_ref):
        g = g_ref[...]
        wqv = wq_ref[0]
        h1 = g.reshape(Qc, K, H1) + wqv[:, None, :]
        h1 = jnp.maximum(h1, 0.0).reshape(Qc * K, H1)
        h2 = jnp.maximum(
            jnp.dot(h1, w2_ref[...], preferred_element_type=jnp.float32)
            + b2_ref[...], 0.0)
        h3 = (jnp.dot(h2, w3_ref[...], preferred_element_type=jnp.float32)
              + b3_ref[...])
        v = v_ref[0]
        h3 = jnp.where(v[:, :, None] > 0.5, h3.reshape(Qc, K, Ho), -jnp.inf)
        o_ref[0] = jnp.max(h3, axis=1)

    return pl.pallas_call(
        body,
        grid=(B, nblk),
        in_specs=[
            pl.BlockSpec((Qc * K, H1), lambda b, q, n=nblk: (b * n + q, 0)),
            pl.BlockSpec((1, Qc, H1), lambda b, q: (b, q, 0)),
            pl.BlockSpec((1, Qc, K), lambda b, q: (b, q, 0)),
            pl.BlockSpec((H1, H2), lambda b, q: (0, 0)),
            pl.BlockSpec((1, H2), lambda b, q: (0, 0)),
            pl.BlockSpec((H2, Ho), lambda b, q: (0, 0)),
            pl.BlockSpec((1, Ho), lambda b, q: (0, 0)),
        ],
        out_specs=pl.BlockSpec((1, Qc, Ho), lambda b, q: (b, q, 0)),
        out_shape=jax.ShapeDtypeStruct((B, n_s, Ho), jnp.float32),
    )(G, wq, val, w2, b2[None, :], w3, b3[None, :])


def kernel(point_cloud_features, point_cloud, params):
    B, N0 = point_cloud.shape[:2]
    feat = point_cloud_features.reshape(B, N0).astype(jnp.int32)
    pid_full = jnp.concatenate(
        [params["pid"][0],
         jnp.zeros((N0 - NUM_ROBOT, FEAT_DIM), jnp.float32)], axis=0)
    x = _embed(feat, params["emb"], pid_full)

    px = point_cloud[..., 0]
    py = point_cloud[..., 1]
    pz = point_cloud[..., 2]

    for mlp_params, r in ((params["sa1"], 0.05),
                          (params["sa2"], 0.3),
                          (params["sa3"], 0.5)):
        (W1, b1), (W2, b2), (W3, b3) = mlp_params
        N = x.shape[1]
        F = x.shape[2]
        n_s = N // 4
        H1 = W1.shape[1]
        if H1 < 128:
            pad = 128 - H1
            W1 = jnp.pad(W1, ((0, 0), (0, pad)))
            b1 = jnp.pad(b1, (0, pad))
            W2 = jnp.pad(W2, ((0, pad), (0, 0)))
            H1 = 128

        R = max(N // 128, 1)
        C = N // R
        qx, qy, qz = _fps(px.reshape(B, R, C), py.reshape(B, R, C),
                          pz.reshape(B, R, C), n_s)
        qx = qx.reshape(B, n_s)
        qy = qy.reshape(B, n_s)
        qz = qz.reshape(B, n_s)

        rows = jnp.concatenate(
            [x.reshape(B * N, F),
             jnp.stack([px, py, pz], axis=-1).reshape(B * N, 3)], axis=1)
        T = _matbias(rows, W1, b1)
        qrows = jnp.stack([qx, qy, qz], axis=-1).reshape(B * n_s, 3)
        wq = _matbias(qrows, -W1[F:], jnp.zeros((H1,), jnp.float32))

        split = 2 if H1 <= 128 else 1
        nbr, val = _select(px, py, pz, qx, qy, qz, r * r, B // split)
        G = _sc_gather(T, nbr.reshape(-1), split)
        x = _mlpmax(G, wq.reshape(B, n_s, H1), val, W2, b2, W3, b3)
        px, py, pz = qx, qy, qz

    pos_out = jnp.stack([px, py, pz], axis=-1)
    return x, pos_out


# speedup vs baseline: 1.0888x; 1.0888x over previous
import functools

import jax
import jax.numpy as jnp
from jax import lax
from jax.experimental import pallas as pl
from jax.experimental.pallas import tpu as pltpu
from jax.experimental.pallas import tpu_sc as plsc

NUM_ROBOT = 2048
FEAT_DIM = 32
K_NBR = 64


def _embed(feat, emb, pid_full):
    B, N = feat.shape
    F = emb.shape[1]

    def body(f_ref, emb_ref, pid_ref, o_ref):
        f = f_ref[0]
        e = emb_ref[...]
        x = pid_ref[...]
        for c in range(3):
            x = x + jnp.where(f == c, 1.0, 0.0) * e[c:c + 1, :]
        o_ref[0] = x

    return pl.pallas_call(
        body,
        grid=(B,),
        in_specs=[
            pl.BlockSpec((1, N, 1), lambda b: (b, 0, 0)),
            pl.BlockSpec((3, F), lambda b: (0, 0)),
            pl.BlockSpec((N, F), lambda b: (0, 0)),
        ],
        out_specs=pl.BlockSpec((1, N, F), lambda b: (b, 0, 0)),
        out_shape=jax.ShapeDtypeStruct((B, N, F), jnp.float32),
    )(feat[..., None], emb, pid_full)


def _fps(px, py, pz, n_s):
    B, R, C = px.shape
    N = R * C
    CS = min(n_s, 128)
    RS = n_s // CS

    def body(px_ref, py_ref, pz_ref, qx_ref, qy_ref, qz_ref):
        pxv = px_ref[...]
        pyv = py_ref[...]
        pzv = pz_ref[...]
        flat = (lax.broadcasted_iota(jnp.int32, (B, R, C), 1) * C
                + lax.broadcasted_iota(jnp.int32, (B, R, C), 2))
        flat_s = (lax.broadcasted_iota(jnp.int32, (B, RS, CS), 1) * CS
                  + lax.broadcasted_iota(jnp.int32, (B, RS, CS), 2))

        def coords(last):
            m = flat == last
            qxs = jnp.sum(jnp.where(m, pxv, 0.0), axis=(1, 2), keepdims=True)
            qys = jnp.sum(jnp.where(m, pyv, 0.0), axis=(1, 2), keepdims=True)
            qzs = jnp.sum(jnp.where(m, pzv, 0.0), axis=(1, 2), keepdims=True)
            return qxs, qys, qzs

        def write_q(slot, qxs, qys, qzs):
            sm = flat_s == slot
            qx_ref[...] = jnp.where(sm, qxs, qx_ref[...])
            qy_ref[...] = jnp.where(sm, qys, qy_ref[...])
            qz_ref[...] = jnp.where(sm, qzs, qz_ref[...])

        def it(i, carry):
            dists, last = carry
            qxs, qys, qzs = coords(last)
            write_q(i - 1, qxs, qys, qzs)
            d = (pxv - qxs) ** 2 + (pyv - qys) ** 2 + (pzv - qzs) ** 2
            dists = jnp.minimum(dists, d)
            mx = jnp.max(dists, axis=(1, 2), keepdims=True)
            nxt = jnp.min(jnp.where(dists == mx, flat, N),
                          axis=(1, 2), keepdims=True)
            return dists, nxt

        dists0 = jnp.full((B, R, C), jnp.inf, jnp.float32)
        last0 = jnp.zeros((B, 1, 1), jnp.int32)
        _, last = lax.fori_loop(1, n_s, it, (dists0, last0))
        qxs, qys, qzs = coords(last)
        write_q(n_s - 1, qxs, qys, qzs)

    out = jax.ShapeDtypeStruct((B, RS, CS), jnp.float32)
    return pl.pallas_call(
        body,
        out_shape=[out, out, out],
    )(px, py, pz)


def _matbias(a, w, b):
    M, Kd = a.shape
    H = w.shape[1]
    RB = min(M, 2048)

    def body(a_ref, w_ref, b_ref, o_ref):
        o_ref[...] = (jnp.dot(a_ref[...], w_ref[...],
                              preferred_element_type=jnp.float32)
                      + b_ref[...])

    return pl.pallas_call(
        body,
        grid=(M // RB,),
        in_specs=[
            pl.BlockSpec((RB, Kd), lambda i: (i, 0)),
            pl.BlockSpec((Kd, H), lambda i: (0, 0)),
            pl.BlockSpec((1, H), lambda i: (0, 0)),
        ],
        out_specs=pl.BlockSpec((RB, H), lambda i: (i, 0)),
        out_shape=jax.ShapeDtypeStruct((M, H), jnp.float32),
    )(a, w, b[None, :])


def _select(px, py, pz, qx, qy, qz, r2, idx_mod, Q=64):
    B, N = px.shape
    n_s = qx.shape[1]
    Q = min(Q, n_s)
    nblk = n_s // Q
    K = K_NBR

    def body(px_ref, py_ref, pz_ref, qx_ref, qy_ref, qz_ref, nbr_ref, val_ref):
        b = pl.program_id(0)
        pxv = px_ref[0]
        pyv = py_ref[0]
        pzv = pz_ref[0]
        qxv = qx_ref[0]
        qyv = qy_ref[0]
        qzv = qz_ref[0]
        d2 = (qxv - pxv) ** 2 + (qyv - pyv) ** 2 + (qzv - pzv) ** 2
        d2 = jnp.where(d2 <= r2, d2, jnp.inf)
        lane = lax.broadcasted_iota(jnp.int32, (Q, N), 1)
        col = lax.broadcasted_iota(jnp.int32, (Q, K), 1)
        cnt = jnp.sum(jnp.where(d2 < jnp.inf, 1, 0), axis=1)
        kmax = jnp.minimum(jnp.max(cnt), K)

        def it(k, carry):
            d2c, nbr, val = carry
            m = jnp.min(d2c, axis=1, keepdims=True)
            idx = jnp.min(jnp.where(d2c == m, lane, N), axis=1, keepdims=True)
            ok = m < jnp.inf
            nbr = jnp.where(col == k, idx, nbr)
            val = jnp.where(col == k, jnp.where(ok, 1.0, 0.0), val)
            d2c = jnp.where(lane == idx, jnp.inf, d2c)
            return d2c, nbr, val

        nbr0 = jnp.zeros((Q, K), jnp.int32)
        val0 = jnp.zeros((Q, K), jnp.float32)
        _, nbr, val = lax.fori_loop(0, kmax, it, (d2, nbr0, val0))
        nbr_ref[0] = nbr + (b % idx_mod) * N
        val_ref[0] = val

    return pl.pallas_call(
        body,
        grid=(B, nblk),
        in_specs=[pl.BlockSpec((1, 1, N), lambda b, q: (b, 0, 0))] * 3
        + [pl.BlockSpec((1, Q, 1), lambda b, q: (b, q, 0))] * 3,
        out_specs=[
            pl.BlockSpec((1, Q, K), lambda b, q: (b, q, 0)),
            pl.BlockSpec((1, Q, K), lambda b, q: (b, q, 0)),
        ],
        out_shape=[
            jax.ShapeDtypeStruct((B, n_s, K), jnp.int32),
            jax.ShapeDtypeStruct((B, n_s, K), jnp.float32),
        ],
    )(px[:, None, :], py[:, None, :], pz[:, None, :],
      qx[..., None], qy[..., None], qz[..., None])


def _sc_gather(table, idx, split):
    V, D = table.shape
    E = idx.shape[0]
    NW = 32
    NB = 2
    epw = E // NW
    chunk = min(epw // NB, 128, max(8, (64 * 1024) // (D * 4)))
    nchunk = epw // chunk
    assert nchunk % NB == 0
    Vs = V // split
    stage = split == 2
    mesh = plsc.VectorSubcoreMesh(core_axis_name="c", subcore_axis_name="s")

    scratch = [
        pltpu.VMEM((epw,), jnp.int32),
        [pltpu.VMEM((chunk, D), jnp.float32) for _ in range(NB)],
        [pltpu.SemaphoreType.DMA for _ in range(NB)],
    ]
    if stage:
        scratch = [pltpu.VMEM_SHARED((Vs, D), jnp.float32)] + scratch

    @functools.partial(
        pl.kernel,
        mesh=mesh,
        out_type=jax.ShapeDtypeStruct((E, D), jnp.float32),
        scratch_types=scratch,
    )
    def k(table_hbm, idx_hbm, out_hbm, *refs):
        if stage:
            sp, idx_v, bufs, sems = refs
        else:
            idx_v, bufs, sems = refs
            sp = table_hbm
        ci = lax.axis_index("c")
        si = lax.axis_index("s")
        if split == 2:
            base = ci * (E // 2) + si * epw
        else:
            base = (si * 2 + ci) * epw

        if stage:
            @pl.when(si == 0)
            def _():
                pltpu.sync_copy(table_hbm.at[pl.ds(ci * Vs, Vs)], sp)

        pltpu.sync_copy(idx_hbm.at[pl.ds(base, epw)], idx_v)
        if stage:
            plsc.subcore_barrier()

        def start(c, b):
            off = pl.multiple_of(c * chunk, 8)
            pltpu.async_copy(sp.at[idx_v.at[pl.ds(off, chunk)]],
                             bufs[b], sems[b])

        def finish(c, b):
            off = pl.multiple_of(c * chunk, 8)
            pltpu.make_async_copy(sp.at[idx_v.at[pl.ds(off, chunk)]],
                                  bufs[b], sems[b]).wait()
            off = pl.multiple_of(base + c * chunk, 8)
            pltpu.sync_copy(bufs[b], out_hbm.at[pl.ds(off, chunk)])

        for b in range(NB):
            start(b, b)

        @pl.loop(NB, nchunk, step=NB)
        def _(c0):
            for b in range(NB):
                finish(c0 - NB + b, b)
                start(c0 + b, b)

        for b in range(NB):
            finish(nchunk - NB + b, b)

    return k(table, idx)


def _mlpmax(G, wq, val, w2, b2, w3, b3, Qc=32):
    B, n_s, H1 = wq.shape
    K = val.shape[2]
    H2 = w2.shape[1]
    Ho = w3.shape[1]
    Qc = min(Qc, n_s)
    nblk = n_s // Qc

    def body(g_ref, wq_ref, v_ref, w2_ref, b2_ref, w3_ref, b3_ref, o_ref):
        g = g_ref[...]
        wqv = wq_ref[0]
        h1 = g.reshape(Qc, K, H1) + wqv[:, None, :]
        h1 = jnp.maximum(h1, 0.0).reshape(Qc * K, H1)
        h2 = jnp.maximum(
            jnp.dot(h1, w2_ref[...], preferred_element_type=jnp.float32)
            + b2_ref[...], 0.0)
        h3 = (jnp.dot(h2, w3_ref[...], preferred_element_type=jnp.float32)
              + b3_ref[...])
        v = v_ref[0]
        h3 = jnp.where(v[:, :, None] > 0.5, h3.reshape(Qc, K, Ho), -jnp.inf)
        o_ref[0] = jnp.max(h3, axis=1)

    return pl.pallas_call(
        body,
        grid=(B, nblk),
        in_specs=[
            pl.BlockSpec((Qc * K, H1), lambda b, q, n=nblk: (b * n + q, 0)),
            pl.BlockSpec((1, Qc, H1), lambda b, q: (b, q, 0)),
            pl.BlockSpec((1, Qc, K), lambda b, q: (b, q, 0)),
            pl.BlockSpec((H1, H2), lambda b, q: (0, 0)),
            pl.BlockSpec((1, H2), lambda b, q: (0, 0)),
            pl.BlockSpec((H2, Ho), lambda b, q: (0, 0)),
            pl.BlockSpec((1, Ho), lambda b, q: (0, 0)),
        ],
        out_specs=pl.BlockSpec((1, Qc, Ho), lambda b, q: (b, q, 0)),
        out_shape=jax.ShapeDtypeStruct((B, n_s, Ho), jnp.float32),
    )(G, wq, val, w2, b2[None, :], w3, b3[None, :])


def kernel(point_cloud_features, point_cloud, params):
    B, N0 = point_cloud.shape[:2]
    feat = point_cloud_features.reshape(B, N0).astype(jnp.int32)
    pid_full = jnp.concatenate(
        [params["pid"][0],
         jnp.zeros((N0 - NUM_ROBOT, FEAT_DIM), jnp.float32)], axis=0)
    x = _embed(feat, params["emb"], pid_full)

    px = point_cloud[..., 0]
    py = point_cloud[..., 1]
    pz = point_cloud[..., 2]

    for mlp_params, r in ((params["sa1"], 0.05),
                          (params["sa2"], 0.3),
                          (params["sa3"], 0.5)):
        (W1, b1), (W2, b2), (W3, b3) = mlp_params
        N = x.shape[1]
        F = x.shape[2]
        n_s = N // 4
        H1 = W1.shape[1]
        if H1 < 128:
            pad = 128 - H1
            W1 = jnp.pad(W1, ((0, 0), (0, pad)))
            b1 = jnp.pad(b1, (0, pad))
            W2 = jnp.pad(W2, ((0, pad), (0, 0)))
            H1 = 128

        R = max(N // 128, 1)
        C = N // R
        qx, qy, qz = _fps(px.reshape(B, R, C), py.reshape(B, R, C),
                          pz.reshape(B, R, C), n_s)
        qx = qx.reshape(B, n_s)
        qy = qy.reshape(B, n_s)
        qz = qz.reshape(B, n_s)

        rows = jnp.concatenate(
            [x.reshape(B * N, F),
             jnp.stack([px, py, pz], axis=-1).reshape(B * N, 3)], axis=1)
        T = _matbias(rows, W1, b1)
        qrows = jnp.stack([qx, qy, qz], axis=-1).reshape(B * n_s, 3)
        wq = _matbias(qrows, -W1[F:], jnp.zeros((H1,), jnp.float32))

        split = 2 if H1 <= 128 else 1
        nbr, val = _select(px, py, pz, qx, qy, qz, r * r, B // split)
        G = _sc_gather(T, nbr.reshape(-1), split)
        x = _mlpmax(G, wq.reshape(B, n_s, H1), val, W2, b2, W3, b3)
        px, py, pz = qx, qy, qz

    pos_out = jnp.stack([px, py, pz], axis=-1)
    return x, pos_out


# speedup vs baseline: 1.1215x; 1.0301x over previous
import functools

import jax
import jax.numpy as jnp
from jax import lax
from jax.experimental import pallas as pl
from jax.experimental.pallas import tpu as pltpu
from jax.experimental.pallas import tpu_sc as plsc

NUM_ROBOT = 2048
FEAT_DIM = 32
K_NBR = 64


def _embed(feat, emb, pid_full):
    B, N = feat.shape
    F = emb.shape[1]

    def body(f_ref, emb_ref, pid_ref, o_ref):
        f = f_ref[0]
        e = emb_ref[...]
        x = pid_ref[...]
        for c in range(3):
            x = x + jnp.where(f == c, 1.0, 0.0) * e[c:c + 1, :]
        o_ref[0] = x

    return pl.pallas_call(
        body,
        grid=(B,),
        in_specs=[
            pl.BlockSpec((1, N, 1), lambda b: (b, 0, 0)),
            pl.BlockSpec((3, F), lambda b: (0, 0)),
            pl.BlockSpec((N, F), lambda b: (0, 0)),
        ],
        out_specs=pl.BlockSpec((1, N, F), lambda b: (b, 0, 0)),
        out_shape=jax.ShapeDtypeStruct((B, N, F), jnp.float32),
    )(feat[..., None], emb, pid_full)


def _fps(px, py, pz, n_s):
    B, R, C = px.shape
    N = R * C
    CS = min(n_s, 128)
    RS = n_s // CS

    def body(px_ref, py_ref, pz_ref, qx_ref, qy_ref, qz_ref):
        pxv = px_ref[...]
        pyv = py_ref[...]
        pzv = pz_ref[...]
        flat = (lax.broadcasted_iota(jnp.int32, (B, R, C), 1) * C
                + lax.broadcasted_iota(jnp.int32, (B, R, C), 2))
        flat_s = (lax.broadcasted_iota(jnp.int32, (B, RS, CS), 1) * CS
                  + lax.broadcasted_iota(jnp.int32, (B, RS, CS), 2))

        def coords(last):
            m = flat == last
            qxs = jnp.sum(jnp.where(m, pxv, 0.0), axis=(1, 2), keepdims=True)
            qys = jnp.sum(jnp.where(m, pyv, 0.0), axis=(1, 2), keepdims=True)
            qzs = jnp.sum(jnp.where(m, pzv, 0.0), axis=(1, 2), keepdims=True)
            return qxs, qys, qzs

        def write_q(slot, qxs, qys, qzs):
            sm = flat_s == slot
            qx_ref[...] = jnp.where(sm, qxs, qx_ref[...])
            qy_ref[...] = jnp.where(sm, qys, qy_ref[...])
            qz_ref[...] = jnp.where(sm, qzs, qz_ref[...])

        def it(i, carry):
            dists, last = carry
            qxs, qys, qzs = coords(last)
            write_q(i - 1, qxs, qys, qzs)
            d = (pxv - qxs) ** 2 + (pyv - qys) ** 2 + (pzv - qzs) ** 2
            dists = jnp.minimum(dists, d)
            mx = jnp.max(dists, axis=(1, 2), keepdims=True)
            nxt = jnp.min(jnp.where(dists == mx, flat, N),
                          axis=(1, 2), keepdims=True)
            return dists, nxt

        dists0 = jnp.full((B, R, C), jnp.inf, jnp.float32)
        last0 = jnp.zeros((B, 1, 1), jnp.int32)
        _, last = lax.fori_loop(1, n_s, it, (dists0, last0))
        qxs, qys, qzs = coords(last)
        write_q(n_s - 1, qxs, qys, qzs)

    out = jax.ShapeDtypeStruct((B, RS, CS), jnp.float32)
    return pl.pallas_call(
        body,
        out_shape=[out, out, out],
    )(px, py, pz)


def _matbias(a, w, b):
    M, Kd = a.shape
    H = w.shape[1]
    RB = min(M, 2048)

    def body(a_ref, w_ref, b_ref, o_ref):
        o_ref[...] = (jnp.dot(a_ref[...], w_ref[...],
                              preferred_element_type=jnp.float32)
                      + b_ref[...])

    return pl.pallas_call(
        body,
        grid=(M // RB,),
        in_specs=[
            pl.BlockSpec((RB, Kd), lambda i: (i, 0)),
            pl.BlockSpec((Kd, H), lambda i: (0, 0)),
            pl.BlockSpec((1, H), lambda i: (0, 0)),
        ],
        out_specs=pl.BlockSpec((RB, H), lambda i: (i, 0)),
        out_shape=jax.ShapeDtypeStruct((M, H), jnp.float32),
    )(a, w, b[None, :])


def _select(px, py, pz, qx, qy, qz, r2, idx_mod, Q=64):
    B, N = px.shape
    n_s = qx.shape[1]
    Q = min(Q, n_s)
    nblk = n_s // Q
    K = K_NBR

    def body(px_ref, py_ref, pz_ref, qx_ref, qy_ref, qz_ref, nbr_ref, val_ref):
        b = pl.program_id(0)
        pxv = px_ref[0]
        pyv = py_ref[0]
        pzv = pz_ref[0]
        qxv = qx_ref[0]
        qyv = qy_ref[0]
        qzv = qz_ref[0]
        d2 = (qxv - pxv) ** 2 + (qyv - pyv) ** 2 + (qzv - pzv) ** 2
        d2 = jnp.where(d2 <= r2, d2, jnp.inf)
        lane = lax.broadcasted_iota(jnp.int32, (Q, N), 1)
        col = lax.broadcasted_iota(jnp.int32, (Q, K), 1)
        cnt = jnp.sum(jnp.where(d2 < jnp.inf, 1, 0), axis=1)
        kmax = jnp.minimum(jnp.max(cnt), K)

        def it(k, carry):
            d2c, nbr, val = carry
            m = jnp.min(d2c, axis=1, keepdims=True)
            idx = jnp.min(jnp.where(d2c == m, lane, N), axis=1, keepdims=True)
            ok = m < jnp.inf
            nbr = jnp.where(col == k, idx, nbr)
            val = jnp.where(col == k, jnp.where(ok, 1.0, 0.0), val)
            d2c = jnp.where(lane == idx, jnp.inf, d2c)
            return d2c, nbr, val

        nbr0 = jnp.zeros((Q, K), jnp.int32)
        val0 = jnp.zeros((Q, K), jnp.float32)
        _, nbr, val = lax.fori_loop(0, kmax, it, (d2, nbr0, val0))
        nbr_ref[0] = nbr + (b % idx_mod) * N
        val_ref[0] = val

    return pl.pallas_call(
        body,
        grid=(B, nblk),
        in_specs=[pl.BlockSpec((1, 1, N), lambda b, q: (b, 0, 0))] * 3
        + [pl.BlockSpec((1, Q, 1), lambda b, q: (b, q, 0))] * 3,
        out_specs=[
            pl.BlockSpec((1, Q, K), lambda b, q: (b, q, 0)),
            pl.BlockSpec((1, Q, K), lambda b, q: (b, q, 0)),
        ],
        out_shape=[
            jax.ShapeDtypeStruct((B, n_s, K), jnp.int32),
            jax.ShapeDtypeStruct((B, n_s, K), jnp.float32),
        ],
    )(px[:, None, :], py[:, None, :], pz[:, None, :],
      qx[..., None], qy[..., None], qz[..., None])


def _sc_gather(table, idx, split):
    V, D = table.shape
    E = idx.shape[0]
    NW = 32
    NB = 2
    epw = E // NW
    chunk = min(epw // NB, 128, max(8, (64 * 1024) // (D * 4)))
    nchunk = epw // chunk
    assert nchunk % NB == 0
    Vs = V // split
    stage = split == 2
    mesh = plsc.VectorSubcoreMesh(core_axis_name="c", subcore_axis_name="s")

    scratch = [
        pltpu.VMEM((epw,), jnp.int32),
        [pltpu.VMEM((chunk, D), jnp.float32) for _ in range(NB)],
        [pltpu.SemaphoreType.DMA for _ in range(NB)],
    ]
    if stage:
        scratch = [pltpu.VMEM_SHARED((Vs, D), jnp.float32)] + scratch

    @functools.partial(
        pl.kernel,
        mesh=mesh,
        out_type=jax.ShapeDtypeStruct((E, D), jnp.float32),
        scratch_types=scratch,
    )
    def k(table_hbm, idx_hbm, out_hbm, *refs):
        if stage:
            sp, idx_v, bufs, sems = refs
        else:
            idx_v, bufs, sems = refs
            sp = table_hbm
        ci = lax.axis_index("c")
        si = lax.axis_index("s")
        if split == 2:
            base = ci * (E // 2) + si * epw
        else:
            base = (si * 2 + ci) * epw

        if stage:
            @pl.when(si == 0)
            def _():
                pltpu.sync_copy(table_hbm.at[pl.ds(ci * Vs, Vs)], sp)

        pltpu.sync_copy(idx_hbm.at[pl.ds(base, epw)], idx_v)
        if stage:
            plsc.subcore_barrier()

        def start(c, b):
            off = pl.multiple_of(c * chunk, 8)
            pltpu.async_copy(sp.at[idx_v.at[pl.ds(off, chunk)]],
                             bufs[b], sems[b])

        def finish(c, b):
            off = pl.multiple_of(c * chunk, 8)
            pltpu.make_async_copy(sp.at[idx_v.at[pl.ds(off, chunk)]],
                                  bufs[b], sems[b]).wait()
            off = pl.multiple_of(base + c * chunk, 8)
            pltpu.sync_copy(bufs[b], out_hbm.at[pl.ds(off, chunk)])

        for b in range(NB):
            start(b, b)

        @pl.loop(NB, nchunk, step=NB)
        def _(c0):
            for b in range(NB):
                finish(c0 - NB + b, b)
                start(c0 + b, b)

        for b in range(NB):
            finish(nchunk - NB + b, b)

    return k(table, idx)


def _mlpmax(G, wq, val, w2, b2, w3, b3, Qc=64):
    B, n_s, H1 = wq.shape
    K = val.shape[2]
    H2 = w2.shape[1]
    Ho = w3.shape[1]
    Qc = min(Qc, n_s)
    nblk = n_s // Qc

    def body(g_ref, wq_ref, v_ref, w2_ref, b2_ref, w3_ref, b3_ref, o_ref):
        g = g_ref[...]
        wqv = wq_ref[0]
        h1 = g.reshape(Qc, K, H1) + wqv[:, None, :]
        h1 = jnp.maximum(h1, 0.0).reshape(Qc * K, H1)
        h2 = jnp.maximum(
            jnp.dot(h1, w2_ref[...], preferred_element_type=jnp.float32)
            + b2_ref[...], 0.0)
        h3 = (jnp.dot(h2, w3_ref[...], preferred_element_type=jnp.float32)
              + b3_ref[...])
        v = v_ref[0]
        h3 = jnp.where(v[:, :, None] > 0.5, h3.reshape(Qc, K, Ho), -jnp.inf)
        o_ref[0] = jnp.max(h3, axis=1)

    return pl.pallas_call(
        body,
        grid=(B, nblk),
        in_specs=[
            pl.BlockSpec((Qc * K, H1), lambda b, q, n=nblk: (b * n + q, 0)),
            pl.BlockSpec((1, Qc, H1), lambda b, q: (b, q, 0)),
            pl.BlockSpec((1, Qc, K), lambda b, q: (b, q, 0)),
            pl.BlockSpec((H1, H2), lambda b, q: (0, 0)),
            pl.BlockSpec((1, H2), lambda b, q: (0, 0)),
            pl.BlockSpec((H2, Ho), lambda b, q: (0, 0)),
            pl.BlockSpec((1, Ho), lambda b, q: (0, 0)),
        ],
        out_specs=pl.BlockSpec((1, Qc, Ho), lambda b, q: (b, q, 0)),
        out_shape=jax.ShapeDtypeStruct((B, n_s, Ho), jnp.float32),
    )(G, wq, val, w2, b2[None, :], w3, b3[None, :])


def kernel(point_cloud_features, point_cloud, params):
    B, N0 = point_cloud.shape[:2]
    feat = point_cloud_features.reshape(B, N0).astype(jnp.int32)
    pid_full = jnp.concatenate(
        [params["pid"][0],
         jnp.zeros((N0 - NUM_ROBOT, FEAT_DIM), jnp.float32)], axis=0)
    x = _embed(feat, params["emb"], pid_full)

    px = point_cloud[..., 0]
    py = point_cloud[..., 1]
    pz = point_cloud[..., 2]

    for mlp_params, r in ((params["sa1"], 0.05),
                          (params["sa2"], 0.3),
                          (params["sa3"], 0.5)):
        (W1, b1), (W2, b2), (W3, b3) = mlp_params
        N = x.shape[1]
        F = x.shape[2]
        n_s = N // 4
        H1 = W1.shape[1]
        if H1 < 128:
            pad = 128 - H1
            W1 = jnp.pad(W1, ((0, 0), (0, pad)))
            b1 = jnp.pad(b1, (0, pad))
            W2 = jnp.pad(W2, ((0, pad), (0, 0)))
            H1 = 128

        R = max(N // 128, 1)
        C = N // R
        qx, qy, qz = _fps(px.reshape(B, R, C), py.reshape(B, R, C),
                          pz.reshape(B, R, C), n_s)
        qx = qx.reshape(B, n_s)
        qy = qy.reshape(B, n_s)
        qz = qz.reshape(B, n_s)

        rows = jnp.concatenate(
            [x.reshape(B * N, F),
             jnp.stack([px, py, pz], axis=-1).reshape(B * N, 3)], axis=1)
        T = _matbias(rows, W1, b1)
        qrows = jnp.stack([qx, qy, qz], axis=-1).reshape(B * n_s, 3)
        wq = _matbias(qrows, -W1[F:], jnp.zeros((H1,), jnp.float32))

        split = 2 if H1 <= 128 else 1
        nbr, val = _select(px, py, pz, qx, qy, qz, r * r, B // split)
        G = _sc_gather(T, nbr.reshape(-1), split)
        x = _mlpmax(G, wq.reshape(B, n_s, H1), val, W2, b2, W3, b3)
        px, py, pz = qx, qy, qz

    pos_out = jnp.stack([px, py, pz], axis=-1)
    return x, pos_out
